# baseline (speedup 1.0000x reference)
import functools

import jax
import jax.numpy as jnp
from jax import lax
from jax.experimental import pallas as pl
from jax.experimental.pallas import tpu as pltpu
from jax.experimental.pallas import tpu_sc as plsc

NUM_NODES = 10000
NUM_EDGES = 320000
DIM = 128
NCORES = 2
NSUB = 16
NWORK = NCORES * NSUB
CHUNK = 128
TOT_CHUNKS = NUM_EDGES // CHUNK
BASE_CHUNKS = TOT_CHUNKS // NWORK
EXTRA = TOT_CHUNKS - BASE_CHUNKS * NWORK
PAD_NODES = 10240
ROWS_PER_SUB = PAD_NODES // NSUB


def _agg_body(with_deg, *refs):
    if with_deg:
        (h_hbm, ei_hbm, zrows_hbm, ones_hbm,
         acc_out, deg_out,
         eidx0, eidx1, rows0, rows1, acc_sh, g0, g1) = refs
    else:
        (h_hbm, ei_hbm, zrows_hbm,
         acc_out,
         eidx0, eidx1, rows0, rows1, acc_sh, g0, g1) = refs

    cid = lax.axis_index("c")
    sid = lax.axis_index("s")
    wid = cid * NSUB + sid
    cstart = wid * BASE_CHUNKS + jnp.minimum(wid, EXTRA)
    nch = BASE_CHUNKS + jnp.where(wid < EXTRA, 1, 0)
    rbase = sid * ROWS_PER_SUB
    rslice = pl.ds(rbase, ROWS_PER_SUB)

    pltpu.sync_copy(zrows_hbm.at[rslice], acc_sh.at[rslice])
    plsc.subcore_barrier()

    def load_idx(c, eref):
        pltpu.sync_copy(ei_hbm.at[:, pl.ds((cstart + c) * CHUNK, CHUNK)], eref)

    def load_idx_async(c, eref, sem):
        pltpu.async_copy(ei_hbm.at[:, pl.ds((cstart + c) * CHUNK, CHUNK)],
                         eref, sem)

    def load_wait(eref, sem):
        pltpu.make_async_copy(ei_hbm.at[:, pl.ds(0, CHUNK)], eref, sem).wait()

    def gather(eref, rows, sem):
        pltpu.async_copy(h_hbm.at[eref.at[0]], rows, sem)

    def gather_wait(eref, rows, sem):
        pltpu.make_async_copy(h_hbm.at[eref.at[0]], rows, sem).wait()

    def scat(rows, eref):
        pltpu.sync_copy(rows, acc_sh.at[eref.at[1]], add=True)

    load_idx(0, eidx0)
    gather(eidx0, rows0, g0)

    @pl.loop(0, nch - 1, step=2)
    def _(i):
        load_idx(i + 1, eidx1)
        gather(eidx1, rows1, g1)
        gather_wait(eidx0, rows0, g0)
        scat(rows0, eidx0)

        @pl.when(i + 2 < nch)
        def _():
            load_idx(i + 2, eidx0)
            gather(eidx0, rows0, g0)

        gather_wait(eidx1, rows1, g1)
        scat(rows1, eidx1)

    @pl.when(nch % 2 == 1)
    def _():
        gather_wait(eidx0, rows0, g0)
        scat(rows0, eidx0)

    plsc.subcore_barrier()
    pltpu.sync_copy(acc_sh.at[rslice], acc_out.at[cid, rslice])

    if with_deg:
        pltpu.sync_copy(zrows_hbm.at[rslice], acc_sh.at[rslice])
        pltpu.sync_copy(ones_hbm, rows0)
        load_idx_async(0, eidx0, g0)
        plsc.subcore_barrier()

        @pl.loop(0, nch - 1, step=2)
        def _(i):
            load_idx_async(i + 1, eidx1, g1)
            load_wait(eidx0, g0)
            pltpu.sync_copy(rows0, acc_sh.at[eidx0.at[1]], add=True)

            @pl.when(i + 2 < nch)
            def _():
                load_idx_async(i + 2, eidx0, g0)

            load_wait(eidx1, g1)
            pltpu.sync_copy(rows0, acc_sh.at[eidx1.at[1]], add=True)

        @pl.when(nch % 2 == 1)
        def _():
            load_wait(eidx0, g0)
            pltpu.sync_copy(rows0, acc_sh.at[eidx0.at[1]], add=True)

        plsc.subcore_barrier()
        pltpu.sync_copy(acc_sh.at[rslice], deg_out.at[cid, rslice])


def _make_agg(with_deg):
    mesh = plsc.VectorSubcoreMesh(core_axis_name="c", subcore_axis_name="s")
    out_type = [jax.ShapeDtypeStruct((NCORES, PAD_NODES, DIM), jnp.float32)]
    scratch = [
        pltpu.VMEM((2, CHUNK), jnp.int32),
        pltpu.VMEM((2, CHUNK), jnp.int32),
        pltpu.VMEM((CHUNK, DIM), jnp.float32),
        pltpu.VMEM((CHUNK, DIM), jnp.float32),
        pltpu.VMEM_SHARED((PAD_NODES, DIM), jnp.float32),
        pltpu.SemaphoreType.DMA,
        pltpu.SemaphoreType.DMA,
    ]
    if with_deg:
        out_type.append(jax.ShapeDtypeStruct((NCORES, PAD_NODES, DIM),
                                             jnp.float32))
    return pl.kernel(functools.partial(_agg_body, with_deg),
                     out_type=out_type, mesh=mesh, scratch_types=scratch)


_agg_with_deg = _make_agg(True)
_agg_no_deg = _make_agg(False)

BN = 1000


def _self_body(h_ref, ws_ref, b_ref, o_ref):
    o_ref[...] = jnp.dot(h_ref[...], ws_ref[...],
                         preferred_element_type=jnp.float32,
                         precision=lax.Precision.HIGHEST) + b_ref[...]


def _self_dense(h, w_self, b):
    grid = (NUM_NODES // BN,)
    row_spec = pl.BlockSpec((BN, DIM), lambda i: (i, 0))
    w_spec = pl.BlockSpec((DIM, DIM), lambda i: (0, 0))
    b_spec = pl.BlockSpec((1, DIM), lambda i: (0, 0))
    return pl.pallas_call(
        _self_body,
        grid=grid,
        in_specs=[row_spec, w_spec, b_spec],
        out_specs=row_spec,
        out_shape=jax.ShapeDtypeStruct((NUM_NODES, DIM), jnp.float32),
    )(h, w_self, b.reshape(1, DIM))


def _rest_body(apply_relu, s_ref, a0_ref, a1_ref, d0_ref, d1_ref,
               wn_ref, o_ref):
    deg = jnp.maximum(d0_ref[0, :, 0:1] + d1_ref[0, :, 0:1], 1.0)
    agg = (a0_ref[0] + a1_ref[0]) / deg
    out = s_ref[...] + jnp.dot(agg, wn_ref[...],
                               preferred_element_type=jnp.float32,
                               precision=lax.Precision.HIGHEST)
    if apply_relu:
        out = jnp.maximum(out, 0.0)
    o_ref[...] = out


def _rest_dense(selfpart, acc, deg, w_neigh, apply_relu):
    grid = (NUM_NODES // BN,)
    row_spec = pl.BlockSpec((BN, DIM), lambda i: (i, 0))
    part0_spec = pl.BlockSpec((1, BN, DIM), lambda i: (0, i, 0))
    part1_spec = pl.BlockSpec((1, BN, DIM), lambda i: (1, i, 0))
    w_spec = pl.BlockSpec((DIM, DIM), lambda i: (0, 0))
    return pl.pallas_call(
        functools.partial(_rest_body, apply_relu),
        grid=grid,
        in_specs=[row_spec, part0_spec, part1_spec, part0_spec, part1_spec,
                  w_spec],
        out_specs=row_spec,
        out_shape=jax.ShapeDtypeStruct((NUM_NODES, DIM), jnp.float32),
    )(selfpart, acc, acc, deg, deg, w_neigh)


def kernel(x, edge_index, W_self1, W_neigh1, b1, W_self2, W_neigh2, b2):
    zrows = jnp.zeros((PAD_NODES, DIM), jnp.float32)
    ones = jnp.ones((CHUNK, DIM), jnp.float32)

    self1 = _self_dense(x, W_self1, b1)
    acc1, deg = _agg_with_deg(x, edge_index, zrows, ones)
    h1 = _rest_dense(self1, acc1, deg, W_neigh1, apply_relu=True)
    self2 = _self_dense(h1, W_self2, b2)
    (acc2,) = _agg_no_deg(h1, edge_index, zrows)
    h2 = _rest_dense(self2, acc2, deg, W_neigh2, apply_relu=False)
    return h2


# speedup vs baseline: 1.0190x; 1.0190x over previous
import functools

import jax
import jax.numpy as jnp
from jax import lax
from jax.experimental import pallas as pl
from jax.experimental.pallas import tpu as pltpu
from jax.experimental.pallas import tpu_sc as plsc

NUM_NODES = 10000
NUM_EDGES = 320000
DIM = 128
NCORES = 2
NSUB = 16
NWORK = NCORES * NSUB
CHUNK = 128
TOT_CHUNKS = NUM_EDGES // CHUNK
BASE_CHUNKS = TOT_CHUNKS // NWORK
EXTRA = TOT_CHUNKS - BASE_CHUNKS * NWORK
PAD_NODES = 10240
ROWS_PER_SUB = PAD_NODES // NSUB


def _agg_body(with_deg, *refs):
    if with_deg:
        (h_hbm, ei_hbm,
         acc_out, deg_out,
         eidx0, eidx1, rows0, rows1, acc_sh, g0, g1) = refs
    else:
        (h_hbm, ei_hbm,
         acc_out,
         eidx0, eidx1, rows0, rows1, acc_sh, g0, g1) = refs

    cid = lax.axis_index("c")
    sid = lax.axis_index("s")
    wid = cid * NSUB + sid
    cstart = wid * BASE_CHUNKS + jnp.minimum(wid, EXTRA)
    nch = BASE_CHUNKS + jnp.where(wid < EXTRA, 1, 0)
    rbase = sid * ROWS_PER_SUB
    rslice = pl.ds(rbase, ROWS_PER_SUB)

    def fill_rows0(value):
        @pl.loop(0, CHUNK)
        def _(r):
            for c in range(DIM // 16):
                rows0[r, pl.ds(c * 16, 16)] = jnp.full((16,), value,
                                                       jnp.float32)

    fill_rows0(0.0)
    for p in range(ROWS_PER_SUB // CHUNK):
        pltpu.sync_copy(rows0, acc_sh.at[pl.ds(rbase + p * CHUNK, CHUNK)])
    plsc.subcore_barrier()

    def load_idx(c, eref):
        pltpu.sync_copy(ei_hbm.at[:, pl.ds((cstart + c) * CHUNK, CHUNK)], eref)

    def load_idx_async(c, eref, sem):
        pltpu.async_copy(ei_hbm.at[:, pl.ds((cstart + c) * CHUNK, CHUNK)],
                         eref, sem)

    def load_wait(eref, sem):
        pltpu.make_async_copy(ei_hbm.at[:, pl.ds(0, CHUNK)], eref, sem).wait()

    def gather(eref, rows, sem):
        pltpu.async_copy(h_hbm.at[eref.at[0]], rows, sem)

    def gather_wait(eref, rows, sem):
        pltpu.make_async_copy(h_hbm.at[eref.at[0]], rows, sem).wait()

    def scat(rows, eref):
        pltpu.sync_copy(rows, acc_sh.at[eref.at[1]], add=True)

    load_idx(0, eidx0)
    gather(eidx0, rows0, g0)

    @pl.loop(0, nch - 1, step=2)
    def _(i):
        load_idx(i + 1, eidx1)
        gather(eidx1, rows1, g1)
        gather_wait(eidx0, rows0, g0)
        scat(rows0, eidx0)

        @pl.when(i + 2 < nch)
        def _():
            load_idx(i + 2, eidx0)
            gather(eidx0, rows0, g0)

        gather_wait(eidx1, rows1, g1)
        scat(rows1, eidx1)

    @pl.when(nch % 2 == 1)
    def _():
        gather_wait(eidx0, rows0, g0)
        scat(rows0, eidx0)

    plsc.subcore_barrier()
    pltpu.sync_copy(acc_sh.at[rslice], acc_out.at[cid, rslice])

    if with_deg:
        @pl.loop(0, CHUNK)
        def _(r):
            for c in range(DIM // 16):
                rows1[r, pl.ds(c * 16, 16)] = jnp.zeros((16,), jnp.float32)

        for p in range(ROWS_PER_SUB // CHUNK):
            pltpu.sync_copy(rows1, acc_sh.at[pl.ds(rbase + p * CHUNK, CHUNK)])
        fill_rows0(1.0)
        load_idx_async(0, eidx0, g0)
        plsc.subcore_barrier()

        @pl.loop(0, nch - 1, step=2)
        def _(i):
            load_idx_async(i + 1, eidx1, g1)
            load_wait(eidx0, g0)
            pltpu.sync_copy(rows0, acc_sh.at[eidx0.at[1]], add=True)

            @pl.when(i + 2 < nch)
            def _():
                load_idx_async(i + 2, eidx0, g0)

            load_wait(eidx1, g1)
            pltpu.sync_copy(rows0, acc_sh.at[eidx1.at[1]], add=True)

        @pl.when(nch % 2 == 1)
        def _():
            load_wait(eidx0, g0)
            pltpu.sync_copy(rows0, acc_sh.at[eidx0.at[1]], add=True)

        plsc.subcore_barrier()
        pltpu.sync_copy(acc_sh.at[rslice], deg_out.at[cid, rslice])


def _make_agg(with_deg):
    mesh = plsc.VectorSubcoreMesh(core_axis_name="c", subcore_axis_name="s")
    out_type = [jax.ShapeDtypeStruct((NCORES, PAD_NODES, DIM), jnp.float32)]
    scratch = [
        pltpu.VMEM((2, CHUNK), jnp.int32),
        pltpu.VMEM((2, CHUNK), jnp.int32),
        pltpu.VMEM((CHUNK, DIM), jnp.float32),
        pltpu.VMEM((CHUNK, DIM), jnp.float32),
        pltpu.VMEM_SHARED((PAD_NODES, DIM), jnp.float32),
        pltpu.SemaphoreType.DMA,
        pltpu.SemaphoreType.DMA,
    ]
    if with_deg:
        out_type.append(jax.ShapeDtypeStruct((NCORES, PAD_NODES, DIM),
                                             jnp.float32))
    return pl.kernel(functools.partial(_agg_body, with_deg),
                     out_type=out_type, mesh=mesh, scratch_types=scratch)


_agg_with_deg = _make_agg(True)
_agg_no_deg = _make_agg(False)

BN = 1000


def _self_body(h_ref, ws_ref, b_ref, o_ref):
    o_ref[...] = jnp.dot(h_ref[...], ws_ref[...],
                         preferred_element_type=jnp.float32,
                         precision=lax.Precision.HIGHEST) + b_ref[...]


def _self_dense(h, w_self, b):
    grid = (NUM_NODES // BN,)
    row_spec = pl.BlockSpec((BN, DIM), lambda i: (i, 0))
    w_spec = pl.BlockSpec((DIM, DIM), lambda i: (0, 0))
    b_spec = pl.BlockSpec((1, DIM), lambda i: (0, 0))
    return pl.pallas_call(
        _self_body,
        grid=grid,
        in_specs=[row_spec, w_spec, b_spec],
        out_specs=row_spec,
        out_shape=jax.ShapeDtypeStruct((NUM_NODES, DIM), jnp.float32),
    )(h, w_self, b.reshape(1, DIM))


def _rest_body(apply_relu, s_ref, a0_ref, a1_ref, d0_ref, d1_ref,
               wn_ref, o_ref):
    deg = jnp.maximum(d0_ref[0, :, 0:1] + d1_ref[0, :, 0:1], 1.0)
    agg = (a0_ref[0] + a1_ref[0]) / deg
    out = s_ref[...] + jnp.dot(agg, wn_ref[...],
                               preferred_element_type=jnp.float32,
                               precision=lax.Precision.HIGHEST)
    if apply_relu:
        out = jnp.maximum(out, 0.0)
    o_ref[...] = out


def _rest_dense(selfpart, acc, deg, w_neigh, apply_relu):
    grid = (NUM_NODES // BN,)
    row_spec = pl.BlockSpec((BN, DIM), lambda i: (i, 0))
    part0_spec = pl.BlockSpec((1, BN, DIM), lambda i: (0, i, 0))
    part1_spec = pl.BlockSpec((1, BN, DIM), lambda i: (1, i, 0))
    w_spec = pl.BlockSpec((DIM, DIM), lambda i: (0, 0))
    return pl.pallas_call(
        functools.partial(_rest_body, apply_relu),
        grid=grid,
        in_specs=[row_spec, part0_spec, part1_spec, part0_spec, part1_spec,
                  w_spec],
        out_specs=row_spec,
        out_shape=jax.ShapeDtypeStruct((NUM_NODES, DIM), jnp.float32),
    )(selfpart, acc, acc, deg, deg, w_neigh)


def kernel(x, edge_index, W_self1, W_neigh1, b1, W_self2, W_neigh2, b2):
    self1 = _self_dense(x, W_self1, b1)
    acc1, deg = _agg_with_deg(x, edge_index)
    h1 = _rest_dense(self1, acc1, deg, W_neigh1, apply_relu=True)
    self2 = _self_dense(h1, W_self2, b2)
    (acc2,) = _agg_no_deg(h1, edge_index)
    h2 = _rest_dense(self2, acc2, deg, W_neigh2, apply_relu=False)
    return h2
